# Initial kernel scaffold; baseline (speedup 1.0000x reference)
import functools

import jax
import jax.numpy as jnp
from jax import lax
from jax.experimental import pallas as pl
from jax.experimental.pallas import tpu as pltpu
from jax.experimental.pallas import tpu_sc as plsc

N = 10000
E = 160000
C = 256
CH = 128
NC = 2
NS = 16
L = 16
CHUNK = 128

EPAD = 163840
EPT32 = EPAD // (NC * NS)
EPT16 = EPAD // NS
NCH32 = EPT32 // CHUNK
NCH16 = EPT16 // CHUNK
NPAD = 10240
RPT = N // NS
RB = 400

_MESH = plsc.VectorSubcoreMesh(
    core_axis_name="c", subcore_axis_name="s", num_cores=NC, num_subcores=NS)


def _fill_zero(ref, rows):
    def body(i, _):
        r = i // 8
        q = i % 8
        ref[r, pl.ds(q * L, L)] = jnp.zeros((L,), jnp.float32)
        return 0
    lax.fori_loop(0, rows * 8, body, 0)


@functools.partial(
    pl.kernel,
    out_type=jax.ShapeDtypeStruct((NC, NPAD), jnp.float32),
    mesh=_MESH,
    scratch_types=[
        pltpu.VMEM((NCH32, CHUNK), jnp.int32),
        pltpu.VMEM((NCH32, CHUNK), jnp.float32),
        pltpu.VMEM((NPAD // NS // 128, 128), jnp.float32),
        pltpu.VMEM_SHARED((NPAD,), jnp.float32),
    ],
)
def _deg_kernel(src_hbm, ew_hbm, out_hbm, idx_v, val_v, zero_v, deg_s):
    c = lax.axis_index("c")
    s = lax.axis_index("s")
    wid = c * NS + s
    rpt = NPAD // NS

    _fill_zero(zero_v, rpt // 128)
    pltpu.sync_copy(zero_v, deg_s.at[pl.ds(s * rpt, rpt)])
    plsc.subcore_barrier()

    pltpu.sync_copy(src_hbm.at[wid], idx_v)
    pltpu.sync_copy(ew_hbm.at[wid], val_v)

    def chunk(j, _):
        pltpu.sync_copy(val_v.at[j], deg_s.at[idx_v.at[j]], add=True)
        return 0
    lax.fori_loop(0, NCH32, chunk, 0)
    plsc.subcore_barrier()
    pltpu.sync_copy(deg_s.at[pl.ds(s * rpt, rpt)],
                    out_hbm.at[c, pl.ds(s * rpt, rpt)])


def _dinv_body(deg_ref, dinv_ref):
    d = deg_ref[0] + deg_ref[1]
    dinv_ref[...] = jnp.where(d > 0, lax.rsqrt(d), 0.0)


_dinv_call = pl.pallas_call(
    _dinv_body,
    out_shape=jax.ShapeDtypeStruct((NPAD // 128, 128), jnp.float32),
)


@functools.partial(
    pl.kernel,
    out_type=jax.ShapeDtypeStruct((NC * NS, NCH32, CHUNK), jnp.float32),
    mesh=_MESH,
    scratch_types=[
        pltpu.VMEM((NCH32, CHUNK), jnp.int32),
        pltpu.VMEM((NCH32, CHUNK), jnp.int32),
        pltpu.VMEM((NCH32, CHUNK), jnp.float32),
        pltpu.VMEM((NCH32, CHUNK), jnp.float32),
        pltpu.VMEM((NPAD,), jnp.float32),
    ],
)
def _wnorm_kernel(src_hbm, dst_hbm, ew_hbm, dinv_hbm, out_hbm,
                  src_v, dst_v, ew_v, w_v, dinv_v):
    c = lax.axis_index("c")
    s = lax.axis_index("s")
    wid = c * NS + s

    pltpu.sync_copy(dinv_hbm, dinv_v)
    pltpu.sync_copy(src_hbm.at[wid], src_v)
    pltpu.sync_copy(dst_hbm.at[wid], dst_v)
    pltpu.sync_copy(ew_hbm.at[wid], ew_v)

    def body(i, _):
        j = i // 8
        q = i % 8
        sv = src_v[j, pl.ds(q * L, L)]
        dv = dst_v[j, pl.ds(q * L, L)]
        ev = ew_v[j, pl.ds(q * L, L)]
        w = plsc.load_gather(dinv_v, [sv]) * ev * plsc.load_gather(dinv_v, [dv])
        w_v[j, pl.ds(q * L, L)] = w
        return 0
    lax.fori_loop(0, NCH32 * 8, body, 0)
    pltpu.sync_copy(w_v, out_hbm.at[wid])


@functools.partial(
    pl.kernel,
    out_type=jax.ShapeDtypeStruct((N, NC, CH), jnp.float32),
    mesh=_MESH,
    scratch_types=[
        pltpu.VMEM((NCH16, CHUNK), jnp.int32),
        pltpu.VMEM((NCH16, CHUNK), jnp.int32),
        pltpu.VMEM((NCH16, CHUNK), jnp.float32),
        pltpu.VMEM((2, CHUNK, CH), jnp.float32),
        pltpu.VMEM((RPT // 5, CH), jnp.float32),
        pltpu.VMEM_SHARED((N, CH), jnp.float32),
        pltpu.SemaphoreType.DMA,
    ],
)
def _smv_kernel(xflat_hbm, srcl_hbm, srch_hbm, dst_hbm, w_hbm, out_hbm,
                src_v, dst_v, w_v, rows_v, zero_v, y_s, sem):
    c = lax.axis_index("c")
    s = lax.axis_index("s")

    _fill_zero(zero_v, RPT // 5)
    for k in range(5):
        pltpu.sync_copy(zero_v,
                        y_s.at[pl.ds(s * RPT + k * (RPT // 5), RPT // 5)])

    @pl.when(c == 0)
    def _():
        pltpu.sync_copy(srcl_hbm.at[s], src_v)

    @pl.when(c == 1)
    def _():
        pltpu.sync_copy(srch_hbm.at[s], src_v)

    pltpu.sync_copy(dst_hbm.at[s], dst_v)
    pltpu.sync_copy(w_hbm.at[s], w_v)
    plsc.subcore_barrier()

    pltpu.make_async_copy(xflat_hbm.at[src_v.at[0]], rows_v.at[0], sem).start()

    def chunk(j, _):
        b = lax.rem(j, 2)

        @pl.when(j < NCH16 - 1)
        def _():
            pltpu.make_async_copy(
                xflat_hbm.at[src_v.at[j + 1]], rows_v.at[1 - b], sem).start()

        pltpu.make_async_copy(
            xflat_hbm.at[src_v.at[j]], rows_v.at[b], sem).wait()

        def edge(e, _):
            wv = lax.broadcast(w_v[j, e], (L,))
            for q in range(8):
                rows_v[b, e, pl.ds(q * L, L)] = \
                    rows_v[b, e, pl.ds(q * L, L)] * wv
            return 0
        lax.fori_loop(0, CHUNK, edge, 0)

        pltpu.sync_copy(rows_v.at[b], y_s.at[dst_v.at[j]], add=True)
        return 0
    lax.fori_loop(0, NCH16, chunk, 0)

    plsc.subcore_barrier()
    pltpu.sync_copy(y_s.at[pl.ds(s * RPT, RPT)],
                    out_hbm.at[pl.ds(s * RPT, RPT), c])


def _dot(a, w):
    return lax.dot_general(a, w, (((1,), (0,)), ((), ())),
                           preferred_element_type=jnp.float32)


def _acc_pair(ref, w_ref, row0):
    lo = ref[:, 0, :]
    hi = ref[:, 1, :]
    return (_dot(lo, w_ref[pl.ds(row0, CH), :]) +
            _dot(hi, w_ref[pl.ds(row0 + CH, CH), :]))


def _stage_a_body(x_ref, sx_ref, ssx_ref, h_ref, sh_ref, ssh_ref,
                  wzr_ref, wp_ref, bzr_ref, bp_ref,
                  z_ref, rh_ref, p_ref):
    X = x_ref[...]
    H = h_ref[...]
    zr = (_dot(X, wzr_ref[pl.ds(0, C), :]) +
          _acc_pair(sx_ref, wzr_ref, 256) +
          _acc_pair(ssx_ref, wzr_ref, 512) +
          _dot(H, wzr_ref[pl.ds(768, C), :]) +
          _acc_pair(sh_ref, wzr_ref, 1024) +
          _acc_pair(ssh_ref, wzr_ref, 1280) +
          bzr_ref[...])
    p = (_dot(X, wp_ref[pl.ds(0, C), :]) +
         _acc_pair(sx_ref, wp_ref, 256) +
         _acc_pair(ssx_ref, wp_ref, 512) +
         bp_ref[...])
    Z = jax.nn.sigmoid(zr[:, :C])
    R = jax.nn.sigmoid(zr[:, C:])
    z_ref[...] = Z
    rh_ref[...] = (R * H).reshape(RB, NC, CH)
    p_ref[...] = p


_stage_a_call = pl.pallas_call(
    _stage_a_body,
    grid=(N // RB,),
    in_specs=[
        pl.BlockSpec((RB, C), lambda i: (i, 0)),
        pl.BlockSpec((RB, NC, CH), lambda i: (i, 0, 0)),
        pl.BlockSpec((RB, NC, CH), lambda i: (i, 0, 0)),
        pl.BlockSpec((RB, C), lambda i: (i, 0)),
        pl.BlockSpec((RB, NC, CH), lambda i: (i, 0, 0)),
        pl.BlockSpec((RB, NC, CH), lambda i: (i, 0, 0)),
        pl.BlockSpec((6 * C, 2 * C), lambda i: (0, 0)),
        pl.BlockSpec((3 * C, C), lambda i: (0, 0)),
        pl.BlockSpec((1, 2 * C), lambda i: (0, 0)),
        pl.BlockSpec((1, C), lambda i: (0, 0)),
    ],
    out_specs=[
        pl.BlockSpec((RB, C), lambda i: (i, 0)),
        pl.BlockSpec((RB, NC, CH), lambda i: (i, 0, 0)),
        pl.BlockSpec((RB, C), lambda i: (i, 0)),
    ],
    out_shape=[
        jax.ShapeDtypeStruct((N, C), jnp.float32),
        jax.ShapeDtypeStruct((N, NC, CH), jnp.float32),
        jax.ShapeDtypeStruct((N, C), jnp.float32),
    ],
)


def _final_body(p_ref, rh_ref, srh_ref, ssrh_ref, wc_ref, bc_ref,
                z_ref, h_ref, out_ref):
    acc = (_acc_pair(rh_ref, wc_ref, 0) +
           _acc_pair(srh_ref, wc_ref, 256) +
           _acc_pair(ssrh_ref, wc_ref, 512) +
           p_ref[...] + bc_ref[...])
    Ht = jnp.tanh(acc)
    Z = z_ref[...]
    out_ref[...] = Z * h_ref[...] + (1.0 - Z) * Ht


_final_call = pl.pallas_call(
    _final_body,
    grid=(N // RB,),
    in_specs=[
        pl.BlockSpec((RB, C), lambda i: (i, 0)),
        pl.BlockSpec((RB, NC, CH), lambda i: (i, 0, 0)),
        pl.BlockSpec((RB, NC, CH), lambda i: (i, 0, 0)),
        pl.BlockSpec((RB, NC, CH), lambda i: (i, 0, 0)),
        pl.BlockSpec((3 * C, C), lambda i: (0, 0)),
        pl.BlockSpec((1, C), lambda i: (0, 0)),
        pl.BlockSpec((RB, C), lambda i: (i, 0)),
        pl.BlockSpec((RB, C), lambda i: (i, 0)),
    ],
    out_specs=pl.BlockSpec((RB, C), lambda i: (i, 0)),
    out_shape=jax.ShapeDtypeStruct((N, C), jnp.float32),
)


def _fold(W):
    return jnp.concatenate([W[0] - W[2], -W[1], 2.0 * W[2]], axis=0)


def kernel(X, edge_index, edge_weight, H,
           W_xz, b_xz, W_hz, b_hz, W_xr, b_xr, W_hr, b_hr,
           W_xh, b_xh, W_hh, b_hh):
    src = edge_index[0]
    dst = edge_index[1]
    pad = EPAD - E
    src_p = jnp.concatenate([src, jnp.zeros((pad,), jnp.int32)])
    dst_p = jnp.concatenate([dst, jnp.zeros((pad,), jnp.int32)])
    ew_p = jnp.concatenate([edge_weight, jnp.zeros((pad,), jnp.float32)])

    src32 = src_p.reshape(NC * NS, NCH32, CHUNK)
    dst32 = dst_p.reshape(NC * NS, NCH32, CHUNK)
    ew32 = ew_p.reshape(NC * NS, NCH32, CHUNK)

    deg2 = _deg_kernel(src32, ew32)
    dinv = _dinv_call(deg2.reshape(NC, NPAD // 128, 128)).reshape(NPAD)
    wn = _wnorm_kernel(src32, dst32, ew32, dinv)

    wn16 = wn.reshape(NS, NCH16, CHUNK)
    srcl = (2 * src_p).reshape(NS, NCH16, CHUNK)
    srch = (2 * src_p + 1).reshape(NS, NCH16, CHUNK)
    dst16 = dst_p.reshape(NS, NCH16, CHUNK)

    def smv(xflat):
        return _smv_kernel(xflat, srcl, srch, dst16, wn16)

    Xf = X.reshape(2 * N, CH)
    Hf = H.reshape(2 * N, CH)
    SX = smv(Xf)
    SSX = smv(SX.reshape(2 * N, CH))
    SH = smv(Hf)
    SSH = smv(SH.reshape(2 * N, CH))

    Wzr = jnp.concatenate([
        jnp.concatenate([_fold(W_xz), _fold(W_hz)], axis=0),
        jnp.concatenate([_fold(W_xr), _fold(W_hr)], axis=0),
    ], axis=1)
    Wp = _fold(W_xh)
    Wc = _fold(W_hh)
    bzr = jnp.concatenate([b_xz + b_hz, b_xr + b_hr]).reshape(1, 2 * C)
    bp = b_xh.reshape(1, C)
    bc = b_hh.reshape(1, C)

    Z, RH2, P = _stage_a_call(X, SX, SSX, H, SH, SSH, Wzr, Wp, bzr, bp)
    SRH = smv(RH2.reshape(2 * N, CH))
    SSRH = smv(SRH.reshape(2 * N, CH))
    return _final_call(P, RH2, SRH, SSRH, Wc, bc, Z, H)


# speedup vs baseline: 2.5702x; 2.5702x over previous
import functools

import jax
import jax.numpy as jnp
from jax import lax
from jax.experimental import pallas as pl
from jax.experimental.pallas import tpu as pltpu
from jax.experimental.pallas import tpu_sc as plsc

N = 10000
E = 160000
C = 256
CH = 128
NC = 2
NS = 16
L = 16
CHUNK = 128

EPAD = 163840
EPT32 = EPAD // (NC * NS)
EPT16 = EPAD // NS
NCH32 = EPT32 // CHUNK
NCH16 = EPT16 // CHUNK
NPAD = 10240
RPT = N // NS
RB = 400

_MESH = plsc.VectorSubcoreMesh(
    core_axis_name="c", subcore_axis_name="s", num_cores=NC, num_subcores=NS)


def _fill_zero(ref, rows, nv):
    def body(i, _):
        r = i // nv
        q = i % nv
        ref[r, pl.ds(q * L, L)] = jnp.zeros((L,), jnp.float32)
        return 0
    lax.fori_loop(0, rows * nv, body, 0)


def _fill_zero_1d(ref, n):
    def body(i, _):
        ref[pl.ds(i * L, L)] = jnp.zeros((L,), jnp.float32)
        return 0
    lax.fori_loop(0, n // L, body, 0)


@functools.partial(
    pl.kernel,
    out_type=jax.ShapeDtypeStruct((NC, NPAD), jnp.float32),
    mesh=_MESH,
    scratch_types=[
        pltpu.VMEM((NCH32, CHUNK), jnp.int32),
        pltpu.VMEM((NCH32, CHUNK), jnp.float32),
        pltpu.VMEM((NPAD // NS,), jnp.float32),
        pltpu.VMEM_SHARED((NPAD,), jnp.float32),
    ],
)
def _deg_kernel(src_hbm, ew_hbm, out_hbm, idx_v, val_v, zero_v, deg_s):
    c = lax.axis_index("c")
    s = lax.axis_index("s")
    wid = c * NS + s
    rpt = NPAD // NS

    _fill_zero_1d(zero_v, rpt)
    pltpu.sync_copy(zero_v, deg_s.at[pl.ds(s * rpt, rpt)])
    plsc.subcore_barrier()

    pltpu.sync_copy(src_hbm.at[wid], idx_v)
    pltpu.sync_copy(ew_hbm.at[wid], val_v)

    def chunk(j, _):
        pltpu.sync_copy(val_v.at[j], deg_s.at[idx_v.at[j]], add=True)
        return 0
    lax.fori_loop(0, NCH32, chunk, 0)
    plsc.subcore_barrier()
    pltpu.sync_copy(deg_s.at[pl.ds(s * rpt, rpt)],
                    out_hbm.at[c, pl.ds(s * rpt, rpt)])


def _dinv_body(deg_ref, dinv_ref):
    d = deg_ref[0] + deg_ref[1]
    dinv_ref[...] = jnp.where(d > 0, lax.rsqrt(d), 0.0)


_dinv_call = pl.pallas_call(
    _dinv_body,
    out_shape=jax.ShapeDtypeStruct((NPAD // 128, 128), jnp.float32),
)


@functools.partial(
    pl.kernel,
    out_type=jax.ShapeDtypeStruct((NC * NS, NCH32, CHUNK), jnp.float32),
    mesh=_MESH,
    scratch_types=[
        pltpu.VMEM((NCH32, CHUNK), jnp.int32),
        pltpu.VMEM((NCH32, CHUNK), jnp.int32),
        pltpu.VMEM((NCH32, CHUNK), jnp.float32),
        pltpu.VMEM((NCH32, CHUNK), jnp.float32),
        pltpu.VMEM((CHUNK,), jnp.float32),
        pltpu.VMEM((CHUNK,), jnp.float32),
    ],
)
def _wnorm_kernel(src_hbm, dst_hbm, ew_hbm, dinv_hbm, out_hbm,
                  src_v, dst_v, ew_v, w_v, ds_v, dd_v):
    c = lax.axis_index("c")
    s = lax.axis_index("s")
    wid = c * NS + s

    pltpu.sync_copy(src_hbm.at[wid], src_v)
    pltpu.sync_copy(dst_hbm.at[wid], dst_v)
    pltpu.sync_copy(ew_hbm.at[wid], ew_v)

    def chunk(j, _):
        pltpu.sync_copy(dinv_hbm.at[src_v.at[j]], ds_v)
        pltpu.sync_copy(dinv_hbm.at[dst_v.at[j]], dd_v)
        for q in range(8):
            sl = pl.ds(q * L, L)
            w_v[j, sl] = ds_v[sl] * ew_v[j, sl] * dd_v[sl]
        return 0
    lax.fori_loop(0, NCH32, chunk, 0)
    pltpu.sync_copy(w_v, out_hbm.at[wid])


MCH = 64
NMC = EPT16 // MCH
IB = 8
NB = NMC // IB


@functools.partial(
    pl.kernel,
    out_type=jax.ShapeDtypeStruct((N, NC, CH), jnp.float32),
    mesh=_MESH,
    scratch_types=[
        pltpu.VMEM((2, IB, MCH), jnp.int32),
        pltpu.VMEM((IB, MCH), jnp.int32),
        pltpu.VMEM((IB, MCH), jnp.float32),
        pltpu.VMEM((2, MCH, CH), jnp.float32),
        pltpu.VMEM_SHARED((N, CH), jnp.float32),
        pltpu.SemaphoreType.DMA,
    ],
)
def _smv_kernel(xflat_hbm, zeros_hbm, srcl_hbm, srch_hbm, dst_hbm, w_hbm,
                out_hbm, src_v, dst_v, w_v, rows_v, y_s, sem):
    c = lax.axis_index("c")
    s = lax.axis_index("s")

    def load_src(bb, buf):
        @pl.when(c == 0)
        def _():
            pltpu.sync_copy(srcl_hbm.at[s, bb], src_v.at[buf])

        @pl.when(c == 1)
        def _():
            pltpu.sync_copy(srch_hbm.at[s, bb], src_v.at[buf])

    load_src(0, 0)
    pltpu.sync_copy(zeros_hbm, y_s.at[pl.ds(s * RPT, RPT)])
    plsc.subcore_barrier()

    pltpu.make_async_copy(
        xflat_hbm.at[src_v.at[0, 0]], rows_v.at[0], sem).start()

    def chunk(j, _):
        bb = j // IB
        pos = lax.rem(j, IB)
        buf = lax.rem(bb, 2)
        b = lax.rem(j, 2)

        @pl.when(pos == 0)
        def _():
            pltpu.sync_copy(dst_hbm.at[s, bb], dst_v)
            pltpu.sync_copy(w_hbm.at[s, bb], w_v)

            @pl.when(bb < NB - 1)
            def _():
                load_src(bb + 1, 1 - buf)

        @pl.when(j < NMC - 1)
        def _():
            jn = j + 1
            pltpu.make_async_copy(
                xflat_hbm.at[src_v.at[lax.rem(jn // IB, 2), lax.rem(jn, IB)]],
                rows_v.at[1 - b], sem).start()

        pltpu.make_async_copy(
            xflat_hbm.at[src_v.at[buf, pos]], rows_v.at[b], sem).wait()

        def group(g, _):
            wv16 = w_v[pos, pl.ds(g * L, L)]
            for e16 in range(L):
                wv = lax.broadcast(wv16[e16], (L,))
                r = g * L + e16
                for u in range(CH // L):
                    rows_v[b, r, pl.ds(u * L, L)] = \
                        rows_v[b, r, pl.ds(u * L, L)] * wv
            return 0
        lax.fori_loop(0, MCH // L, group, 0)

        pltpu.sync_copy(rows_v.at[b], y_s.at[dst_v.at[pos]], add=True)
        return 0
    lax.fori_loop(0, NMC, chunk, 0)

    plsc.subcore_barrier()
    pltpu.sync_copy(y_s.at[pl.ds(s * RPT, RPT)],
                    out_hbm.at[pl.ds(s * RPT, RPT), c])


def _dot(a, w):
    return lax.dot_general(a, w, (((1,), (0,)), ((), ())),
                           preferred_element_type=jnp.float32)


def _acc_pair(ref, w_ref, row0):
    return (_dot(ref[:, 0, :], w_ref[pl.ds(row0, CH), :]) +
            _dot(ref[:, 1, :], w_ref[pl.ds(row0 + CH, CH), :]))


def _stage_a_body(x_ref, sx_ref, ssx_ref, h_ref, sh_ref, ssh_ref,
                  wzr_ref, wp_ref, bzr_ref, bp_ref,
                  z_ref, rh_ref, p_ref):
    X = x_ref[...]
    H = h_ref[...]
    zr = (_dot(X, wzr_ref[pl.ds(0, C), :]) +
          _acc_pair(sx_ref, wzr_ref, 256) +
          _acc_pair(ssx_ref, wzr_ref, 512) +
          _dot(H, wzr_ref[pl.ds(768, C), :]) +
          _acc_pair(sh_ref, wzr_ref, 1024) +
          _acc_pair(ssh_ref, wzr_ref, 1280) +
          bzr_ref[...])
    p = (_dot(X, wp_ref[pl.ds(0, C), :]) +
         _acc_pair(sx_ref, wp_ref, 256) +
         _acc_pair(ssx_ref, wp_ref, 512) +
         bp_ref[...])
    Z = jax.nn.sigmoid(zr[:, :C])
    R = jax.nn.sigmoid(zr[:, C:])
    z_ref[...] = Z
    rh_ref[...] = (R * H).reshape(RB, NC, CH)
    p_ref[...] = p


_stage_a_call = pl.pallas_call(
    _stage_a_body,
    grid=(N // RB,),
    in_specs=[
        pl.BlockSpec((RB, C), lambda i: (i, 0)),
        pl.BlockSpec((RB, NC, CH), lambda i: (i, 0, 0)),
        pl.BlockSpec((RB, NC, CH), lambda i: (i, 0, 0)),
        pl.BlockSpec((RB, C), lambda i: (i, 0)),
        pl.BlockSpec((RB, NC, CH), lambda i: (i, 0, 0)),
        pl.BlockSpec((RB, NC, CH), lambda i: (i, 0, 0)),
        pl.BlockSpec((6 * C, 2 * C), lambda i: (0, 0)),
        pl.BlockSpec((3 * C, C), lambda i: (0, 0)),
        pl.BlockSpec((1, 2 * C), lambda i: (0, 0)),
        pl.BlockSpec((1, C), lambda i: (0, 0)),
    ],
    out_specs=[
        pl.BlockSpec((RB, C), lambda i: (i, 0)),
        pl.BlockSpec((RB, NC, CH), lambda i: (i, 0, 0)),
        pl.BlockSpec((RB, C), lambda i: (i, 0)),
    ],
    out_shape=[
        jax.ShapeDtypeStruct((N, C), jnp.float32),
        jax.ShapeDtypeStruct((N, NC, CH), jnp.float32),
        jax.ShapeDtypeStruct((N, C), jnp.float32),
    ],
)


def _final_body(p_ref, rh_ref, srh_ref, ssrh_ref, wc_ref, bc_ref,
                z_ref, h_ref, out_ref):
    acc = (_acc_pair(rh_ref, wc_ref, 0) +
           _acc_pair(srh_ref, wc_ref, 256) +
           _acc_pair(ssrh_ref, wc_ref, 512) +
           p_ref[...] + bc_ref[...])
    Ht = jnp.tanh(acc)
    Z = z_ref[...]
    out_ref[...] = Z * h_ref[...] + (1.0 - Z) * Ht


_final_call = pl.pallas_call(
    _final_body,
    grid=(N // RB,),
    in_specs=[
        pl.BlockSpec((RB, C), lambda i: (i, 0)),
        pl.BlockSpec((RB, NC, CH), lambda i: (i, 0, 0)),
        pl.BlockSpec((RB, NC, CH), lambda i: (i, 0, 0)),
        pl.BlockSpec((RB, NC, CH), lambda i: (i, 0, 0)),
        pl.BlockSpec((3 * C, C), lambda i: (0, 0)),
        pl.BlockSpec((1, C), lambda i: (0, 0)),
        pl.BlockSpec((RB, C), lambda i: (i, 0)),
        pl.BlockSpec((RB, C), lambda i: (i, 0)),
    ],
    out_specs=pl.BlockSpec((RB, C), lambda i: (i, 0)),
    out_shape=jax.ShapeDtypeStruct((N, C), jnp.float32),
)


def _fold(W):
    return jnp.concatenate([W[0] - W[2], -W[1], 2.0 * W[2]], axis=0)


def kernel(X, edge_index, edge_weight, H,
           W_xz, b_xz, W_hz, b_hz, W_xr, b_xr, W_hr, b_hr,
           W_xh, b_xh, W_hh, b_hh):
    src = edge_index[0]
    dst = edge_index[1]
    pad = EPAD - E
    src_p = jnp.concatenate([src, jnp.zeros((pad,), jnp.int32)])
    dst_p = jnp.concatenate([dst, jnp.zeros((pad,), jnp.int32)])
    ew_p = jnp.concatenate([edge_weight, jnp.zeros((pad,), jnp.float32)])

    src32 = src_p.reshape(NC * NS, NCH32, CHUNK)
    dst32 = dst_p.reshape(NC * NS, NCH32, CHUNK)
    ew32 = ew_p.reshape(NC * NS, NCH32, CHUNK)

    deg2 = _deg_kernel(src32, ew32)
    dinv = _dinv_call(deg2.reshape(NC, NPAD // 128, 128)).reshape(NPAD)
    wn = _wnorm_kernel(src32, dst32, ew32, dinv)

    wn16 = wn.reshape(NS, NB, IB, MCH)
    srcl = (2 * src_p).reshape(NS, NB, IB, MCH)
    srch = (2 * src_p + 1).reshape(NS, NB, IB, MCH)
    dst16 = dst_p.reshape(NS, NB, IB, MCH)
    zeros = jnp.zeros((RPT, CH), jnp.float32)

    def smv(xflat):
        return _smv_kernel(xflat, zeros, srcl, srch, dst16, wn16)

    Xf = X.reshape(2 * N, CH)
    Hf = H.reshape(2 * N, CH)
    SX = smv(Xf)
    SSX = smv(SX.reshape(2 * N, CH))
    SH = smv(Hf)
    SSH = smv(SH.reshape(2 * N, CH))

    Wzr = jnp.concatenate([
        jnp.concatenate([_fold(W_xz), _fold(W_hz)], axis=0),
        jnp.concatenate([_fold(W_xr), _fold(W_hr)], axis=0),
    ], axis=1)
    Wp = _fold(W_xh)
    Wc = _fold(W_hh)
    bzr = jnp.concatenate([b_xz + b_hz, b_xr + b_hr]).reshape(1, 2 * C)
    bp = b_xh.reshape(1, C)
    bc = b_hh.reshape(1, C)

    Z, RH2, P = _stage_a_call(X, SX, SSX, H, SH, SSH, Wzr, Wp, bzr, bp)
    SRH = smv(RH2.reshape(2 * N, CH))
    SSRH = smv(SRH.reshape(2 * N, CH))
    return _final_call(P, RH2, SRH, SSRH, Wc, bc, Z, H)


# speedup vs baseline: 2.7392x; 1.0658x over previous
import functools

import jax
import jax.numpy as jnp
from jax import lax
from jax.experimental import pallas as pl
from jax.experimental.pallas import tpu as pltpu
from jax.experimental.pallas import tpu_sc as plsc

N = 10000
E = 160000
C = 256
CH = 128
NC = 2
NS = 16
L = 16
CHUNK = 128

EPAD = 163840
EPT32 = EPAD // (NC * NS)
EPT16 = EPAD // NS
NCH32 = EPT32 // CHUNK
NCH16 = EPT16 // CHUNK
NPAD = 10240
RPT = N // NS
RB = 400

_MESH = plsc.VectorSubcoreMesh(
    core_axis_name="c", subcore_axis_name="s", num_cores=NC, num_subcores=NS)


def _fill_zero(ref, rows, nv):
    def body(i, _):
        r = i // nv
        q = i % nv
        ref[r, pl.ds(q * L, L)] = jnp.zeros((L,), jnp.float32)
        return 0
    lax.fori_loop(0, rows * nv, body, 0)


def _fill_zero_1d(ref, n):
    def body(i, _):
        ref[pl.ds(i * L, L)] = jnp.zeros((L,), jnp.float32)
        return 0
    lax.fori_loop(0, n // L, body, 0)


@functools.partial(
    pl.kernel,
    out_type=jax.ShapeDtypeStruct((NC, NPAD), jnp.float32),
    mesh=_MESH,
    scratch_types=[
        pltpu.VMEM((NCH32, CHUNK), jnp.int32),
        pltpu.VMEM((NCH32, CHUNK), jnp.float32),
        pltpu.VMEM((NPAD // NS,), jnp.float32),
        pltpu.VMEM_SHARED((NPAD,), jnp.float32),
    ],
)
def _deg_kernel(src_hbm, ew_hbm, out_hbm, idx_v, val_v, zero_v, deg_s):
    c = lax.axis_index("c")
    s = lax.axis_index("s")
    wid = c * NS + s
    rpt = NPAD // NS

    _fill_zero_1d(zero_v, rpt)
    pltpu.sync_copy(zero_v, deg_s.at[pl.ds(s * rpt, rpt)])
    plsc.subcore_barrier()

    pltpu.sync_copy(src_hbm.at[wid], idx_v)
    pltpu.sync_copy(ew_hbm.at[wid], val_v)

    def chunk(j, _):
        pltpu.sync_copy(val_v.at[j], deg_s.at[idx_v.at[j]], add=True)
        return 0
    lax.fori_loop(0, NCH32, chunk, 0)
    plsc.subcore_barrier()
    pltpu.sync_copy(deg_s.at[pl.ds(s * rpt, rpt)],
                    out_hbm.at[c, pl.ds(s * rpt, rpt)])


def _dinv_body(deg_ref, dinv_ref):
    d = deg_ref[0] + deg_ref[1]
    dinv_ref[...] = jnp.where(d > 0, lax.rsqrt(d), 0.0)


_dinv_call = pl.pallas_call(
    _dinv_body,
    out_shape=jax.ShapeDtypeStruct((NPAD // 128, 128), jnp.float32),
)


@functools.partial(
    pl.kernel,
    out_type=jax.ShapeDtypeStruct((NC * NS, NCH32, CHUNK), jnp.float32),
    mesh=_MESH,
    scratch_types=[
        pltpu.VMEM((NCH32, CHUNK), jnp.int32),
        pltpu.VMEM((NCH32, CHUNK), jnp.int32),
        pltpu.VMEM((NCH32, CHUNK), jnp.float32),
        pltpu.VMEM((NCH32, CHUNK), jnp.float32),
        pltpu.VMEM((CHUNK,), jnp.float32),
        pltpu.VMEM((CHUNK,), jnp.float32),
    ],
)
def _wnorm_kernel(src_hbm, dst_hbm, ew_hbm, dinv_hbm, out_hbm,
                  src_v, dst_v, ew_v, w_v, ds_v, dd_v):
    c = lax.axis_index("c")
    s = lax.axis_index("s")
    wid = c * NS + s

    pltpu.sync_copy(src_hbm.at[wid], src_v)
    pltpu.sync_copy(dst_hbm.at[wid], dst_v)
    pltpu.sync_copy(ew_hbm.at[wid], ew_v)

    def chunk(j, _):
        pltpu.sync_copy(dinv_hbm.at[src_v.at[j]], ds_v)
        pltpu.sync_copy(dinv_hbm.at[dst_v.at[j]], dd_v)
        for q in range(8):
            sl = pl.ds(q * L, L)
            w_v[j, sl] = ds_v[sl] * ew_v[j, sl] * dd_v[sl]
        return 0
    lax.fori_loop(0, NCH32, chunk, 0)
    pltpu.sync_copy(w_v, out_hbm.at[wid])


MCH = 32
NMC = EPT16 // MCH
IB = 8
NB = NMC // IB
RING = 4


@functools.partial(
    pl.kernel,
    out_type=jax.ShapeDtypeStruct((N, NC, CH), jnp.float32),
    mesh=_MESH,
    scratch_types=[
        pltpu.VMEM((2, IB, MCH), jnp.int32),
        pltpu.VMEM((2, IB, MCH), jnp.int32),
        pltpu.VMEM((2, IB, MCH), jnp.float32),
        pltpu.VMEM((RING, MCH, CH), jnp.float32),
        pltpu.VMEM_SHARED((N, CH), jnp.float32),
        pltpu.SemaphoreType.DMA,
        pltpu.SemaphoreType.DMA,
        pltpu.SemaphoreType.DMA,
        pltpu.SemaphoreType.DMA,
    ],
)
def _smv_kernel(xflat_hbm, zeros_hbm, srcl_hbm, srch_hbm, dst_hbm, w_hbm,
                out_hbm, src_v, dst_v, w_v, rows_v, y_s,
                sem_g, sem_s, sem_src, sem_dw):
    c = lax.axis_index("c")
    s = lax.axis_index("s")

    def start_idx(bb, buf):
        @pl.when(c == 0)
        def _():
            pltpu.async_copy(srcl_hbm.at[s, bb], src_v.at[buf], sem_src)

        @pl.when(c == 1)
        def _():
            pltpu.async_copy(srch_hbm.at[s, bb], src_v.at[buf], sem_src)

        pltpu.async_copy(dst_hbm.at[s, bb], dst_v.at[buf], sem_dw)
        pltpu.async_copy(w_hbm.at[s, bb], w_v.at[buf], sem_dw)

    def wait_src():
        pltpu.make_async_copy(
            srcl_hbm.at[s, 0], src_v.at[0], sem_src).wait()

    def wait_dw():
        pltpu.make_async_copy(dst_hbm.at[s, 0], dst_v.at[0], sem_dw).wait()
        pltpu.make_async_copy(w_hbm.at[s, 0], w_v.at[0], sem_dw).wait()

    def start_gather(j):
        buf = lax.rem(j // IB, 2)
        pltpu.async_copy(xflat_hbm.at[src_v.at[buf, lax.rem(j, IB)]],
                         rows_v.at[lax.rem(j, RING)], sem_g)

    def wait_gather():
        pltpu.make_async_copy(
            xflat_hbm.at[src_v.at[0, 0]], rows_v.at[0], sem_g).wait()

    def start_scatter(j):
        buf = lax.rem(j // IB, 2)
        pltpu.async_copy(rows_v.at[lax.rem(j, RING)],
                         y_s.at[dst_v.at[buf, lax.rem(j, IB)]],
                         sem_s, add=True)

    def wait_scatter():
        pltpu.make_async_copy(
            rows_v.at[0], y_s.at[dst_v.at[0, 0]], sem_s).wait()

    @pl.when(c == 0)
    def _():
        pltpu.sync_copy(srcl_hbm.at[s, 0], src_v.at[0])

    @pl.when(c == 1)
    def _():
        pltpu.sync_copy(srch_hbm.at[s, 0], src_v.at[0])

    pltpu.sync_copy(dst_hbm.at[s, 0], dst_v.at[0])
    pltpu.sync_copy(w_hbm.at[s, 0], w_v.at[0])
    start_gather(0)
    start_gather(1)
    pltpu.sync_copy(zeros_hbm, y_s.at[pl.ds(s * RPT, RPT)])
    plsc.subcore_barrier()

    def chunk(j, _):
        bb = j // IB
        pos = lax.rem(j, IB)
        buf = lax.rem(bb, 2)
        r = lax.rem(j, RING)

        @pl.when(j >= 2)
        def _():
            wait_scatter()

        @pl.when(jnp.logical_and(pos == 0, bb >= 1))
        def _():
            wait_dw()

        @pl.when(jnp.logical_and(pos == 1, bb < NB - 1))
        def _():
            start_idx(bb + 1, 1 - buf)

        @pl.when(jnp.logical_and(pos == IB - 2, bb < NB - 1))
        def _():
            wait_src()

        @pl.when(j < NMC - 2)
        def _():
            start_gather(j + 2)

        wait_gather()

        def group(g, _):
            wv16 = w_v[buf, pos, pl.ds(g * L, L)]
            for e16 in range(L):
                wv = lax.broadcast(wv16[e16], (L,))
                row = g * L + e16
                for u in range(CH // L):
                    rows_v[r, row, pl.ds(u * L, L)] = \
                        rows_v[r, row, pl.ds(u * L, L)] * wv
            return 0
        lax.fori_loop(0, MCH // L, group, 0)

        start_scatter(j)
        return 0
    lax.fori_loop(0, NMC, chunk, 0)

    wait_scatter()
    wait_scatter()
    plsc.subcore_barrier()
    pltpu.sync_copy(y_s.at[pl.ds(s * RPT, RPT)],
                    out_hbm.at[pl.ds(s * RPT, RPT), c])


def _dot(a, w):
    return lax.dot_general(a, w, (((1,), (0,)), ((), ())),
                           preferred_element_type=jnp.float32)


def _acc_pair(ref, w_ref, row0):
    return (_dot(ref[:, 0, :], w_ref[pl.ds(row0, CH), :]) +
            _dot(ref[:, 1, :], w_ref[pl.ds(row0 + CH, CH), :]))


def _stage_a_body(x_ref, sx_ref, ssx_ref, h_ref, sh_ref, ssh_ref,
                  wzr_ref, wp_ref, bzr_ref, bp_ref,
                  z_ref, rh_ref, p_ref):
    X = x_ref[...]
    H = h_ref[...]
    zr = (_dot(X, wzr_ref[pl.ds(0, C), :]) +
          _acc_pair(sx_ref, wzr_ref, 256) +
          _acc_pair(ssx_ref, wzr_ref, 512) +
          _dot(H, wzr_ref[pl.ds(768, C), :]) +
          _acc_pair(sh_ref, wzr_ref, 1024) +
          _acc_pair(ssh_ref, wzr_ref, 1280) +
          bzr_ref[...])
    p = (_dot(X, wp_ref[pl.ds(0, C), :]) +
         _acc_pair(sx_ref, wp_ref, 256) +
         _acc_pair(ssx_ref, wp_ref, 512) +
         bp_ref[...])
    Z = jax.nn.sigmoid(zr[:, :C])
    R = jax.nn.sigmoid(zr[:, C:])
    z_ref[...] = Z
    rh_ref[...] = (R * H).reshape(RB, NC, CH)
    p_ref[...] = p


_stage_a_call = pl.pallas_call(
    _stage_a_body,
    grid=(N // RB,),
    in_specs=[
        pl.BlockSpec((RB, C), lambda i: (i, 0)),
        pl.BlockSpec((RB, NC, CH), lambda i: (i, 0, 0)),
        pl.BlockSpec((RB, NC, CH), lambda i: (i, 0, 0)),
        pl.BlockSpec((RB, C), lambda i: (i, 0)),
        pl.BlockSpec((RB, NC, CH), lambda i: (i, 0, 0)),
        pl.BlockSpec((RB, NC, CH), lambda i: (i, 0, 0)),
        pl.BlockSpec((6 * C, 2 * C), lambda i: (0, 0)),
        pl.BlockSpec((3 * C, C), lambda i: (0, 0)),
        pl.BlockSpec((1, 2 * C), lambda i: (0, 0)),
        pl.BlockSpec((1, C), lambda i: (0, 0)),
    ],
    out_specs=[
        pl.BlockSpec((RB, C), lambda i: (i, 0)),
        pl.BlockSpec((RB, NC, CH), lambda i: (i, 0, 0)),
        pl.BlockSpec((RB, C), lambda i: (i, 0)),
    ],
    out_shape=[
        jax.ShapeDtypeStruct((N, C), jnp.float32),
        jax.ShapeDtypeStruct((N, NC, CH), jnp.float32),
        jax.ShapeDtypeStruct((N, C), jnp.float32),
    ],
)


def _final_body(p_ref, rh_ref, srh_ref, ssrh_ref, wc_ref, bc_ref,
                z_ref, h_ref, out_ref):
    acc = (_acc_pair(rh_ref, wc_ref, 0) +
           _acc_pair(srh_ref, wc_ref, 256) +
           _acc_pair(ssrh_ref, wc_ref, 512) +
           p_ref[...] + bc_ref[...])
    Ht = jnp.tanh(acc)
    Z = z_ref[...]
    out_ref[...] = Z * h_ref[...] + (1.0 - Z) * Ht


_final_call = pl.pallas_call(
    _final_body,
    grid=(N // RB,),
    in_specs=[
        pl.BlockSpec((RB, C), lambda i: (i, 0)),
        pl.BlockSpec((RB, NC, CH), lambda i: (i, 0, 0)),
        pl.BlockSpec((RB, NC, CH), lambda i: (i, 0, 0)),
        pl.BlockSpec((RB, NC, CH), lambda i: (i, 0, 0)),
        pl.BlockSpec((3 * C, C), lambda i: (0, 0)),
        pl.BlockSpec((1, C), lambda i: (0, 0)),
        pl.BlockSpec((RB, C), lambda i: (i, 0)),
        pl.BlockSpec((RB, C), lambda i: (i, 0)),
    ],
    out_specs=pl.BlockSpec((RB, C), lambda i: (i, 0)),
    out_shape=jax.ShapeDtypeStruct((N, C), jnp.float32),
)


def _fold(W):
    return jnp.concatenate([W[0] - W[2], -W[1], 2.0 * W[2]], axis=0)


def kernel(X, edge_index, edge_weight, H,
           W_xz, b_xz, W_hz, b_hz, W_xr, b_xr, W_hr, b_hr,
           W_xh, b_xh, W_hh, b_hh):
    src = edge_index[0]
    dst = edge_index[1]
    pad = EPAD - E
    src_p = jnp.concatenate([src, jnp.zeros((pad,), jnp.int32)])
    dst_p = jnp.concatenate([dst, jnp.zeros((pad,), jnp.int32)])
    ew_p = jnp.concatenate([edge_weight, jnp.zeros((pad,), jnp.float32)])

    src32 = src_p.reshape(NC * NS, NCH32, CHUNK)
    dst32 = dst_p.reshape(NC * NS, NCH32, CHUNK)
    ew32 = ew_p.reshape(NC * NS, NCH32, CHUNK)

    deg2 = _deg_kernel(src32, ew32)
    dinv = _dinv_call(deg2.reshape(NC, NPAD // 128, 128)).reshape(NPAD)
    wn = _wnorm_kernel(src32, dst32, ew32, dinv)

    wn16 = wn.reshape(NS, NB, IB, MCH)
    srcl = (2 * src_p).reshape(NS, NB, IB, MCH)
    srch = (2 * src_p + 1).reshape(NS, NB, IB, MCH)
    dst16 = dst_p.reshape(NS, NB, IB, MCH)
    zeros = jnp.zeros((RPT, CH), jnp.float32)

    def smv(xflat):
        return _smv_kernel(xflat, zeros, srcl, srch, dst16, wn16)

    Xf = X.reshape(2 * N, CH)
    Hf = H.reshape(2 * N, CH)
    SX = smv(Xf)
    SSX = smv(SX.reshape(2 * N, CH))
    SH = smv(Hf)
    SSH = smv(SH.reshape(2 * N, CH))

    Wzr = jnp.concatenate([
        jnp.concatenate([_fold(W_xz), _fold(W_hz)], axis=0),
        jnp.concatenate([_fold(W_xr), _fold(W_hr)], axis=0),
    ], axis=1)
    Wp = _fold(W_xh)
    Wc = _fold(W_hh)
    bzr = jnp.concatenate([b_xz + b_hz, b_xr + b_hr]).reshape(1, 2 * C)
    bp = b_xh.reshape(1, C)
    bc = b_hh.reshape(1, C)

    Z, RH2, P = _stage_a_call(X, SX, SSX, H, SH, SSH, Wzr, Wp, bzr, bp)
    SRH = smv(RH2.reshape(2 * N, CH))
    SSRH = smv(SRH.reshape(2 * N, CH))
    return _final_call(P, RH2, SRH, SSRH, Wc, bc, Z, H)


# speedup vs baseline: 2.9174x; 1.0650x over previous
import functools

import jax
import jax.numpy as jnp
from jax import lax
from jax.experimental import pallas as pl
from jax.experimental.pallas import tpu as pltpu
from jax.experimental.pallas import tpu_sc as plsc

N = 10000
E = 160000
C = 256
CH = 128
NC = 2
NS = 16
L = 16
CHUNK = 128

EPAD = 163840
EPT32 = EPAD // (NC * NS)
EPT16 = EPAD // NS
NCH32 = EPT32 // CHUNK
NCH16 = EPT16 // CHUNK
NPAD = 10240
RPT = N // NS
RB = 400

_MESH = plsc.VectorSubcoreMesh(
    core_axis_name="c", subcore_axis_name="s", num_cores=NC, num_subcores=NS)


def _fill_zero(ref, rows, nv):
    def body(i, _):
        r = i // nv
        q = i % nv
        ref[r, pl.ds(q * L, L)] = jnp.zeros((L,), jnp.float32)
        return 0
    lax.fori_loop(0, rows * nv, body, 0)


def _fill_zero_1d(ref, n):
    def body(i, _):
        ref[pl.ds(i * L, L)] = jnp.zeros((L,), jnp.float32)
        return 0
    lax.fori_loop(0, n // L, body, 0)


@functools.partial(
    pl.kernel,
    out_type=jax.ShapeDtypeStruct((NC, NPAD), jnp.float32),
    mesh=_MESH,
    scratch_types=[
        pltpu.VMEM((NCH32, CHUNK), jnp.int32),
        pltpu.VMEM((NCH32, CHUNK), jnp.float32),
        pltpu.VMEM((NPAD // NS,), jnp.float32),
        pltpu.VMEM_SHARED((NPAD,), jnp.float32),
    ],
)
def _deg_kernel(src_hbm, ew_hbm, out_hbm, idx_v, val_v, zero_v, deg_s):
    c = lax.axis_index("c")
    s = lax.axis_index("s")
    wid = c * NS + s
    rpt = NPAD // NS

    _fill_zero_1d(zero_v, rpt)
    pltpu.sync_copy(zero_v, deg_s.at[pl.ds(s * rpt, rpt)])
    plsc.subcore_barrier()

    pltpu.sync_copy(src_hbm.at[wid], idx_v)
    pltpu.sync_copy(ew_hbm.at[wid], val_v)

    def chunk(j, _):
        pltpu.sync_copy(val_v.at[j], deg_s.at[idx_v.at[j]], add=True)
        return 0
    lax.fori_loop(0, NCH32, chunk, 0)
    plsc.subcore_barrier()
    pltpu.sync_copy(deg_s.at[pl.ds(s * rpt, rpt)],
                    out_hbm.at[c, pl.ds(s * rpt, rpt)])


def _dinv_body(deg_ref, dinv_ref):
    d = deg_ref[0] + deg_ref[1]
    dinv_ref[...] = jnp.where(d > 0, lax.rsqrt(d), 0.0)


_dinv_call = pl.pallas_call(
    _dinv_body,
    out_shape=jax.ShapeDtypeStruct((NPAD // 128, 128), jnp.float32),
)


@functools.partial(
    pl.kernel,
    out_type=jax.ShapeDtypeStruct((NC * NS, NCH32, CHUNK), jnp.float32),
    mesh=_MESH,
    scratch_types=[
        pltpu.VMEM((NCH32, CHUNK), jnp.int32),
        pltpu.VMEM((NCH32, CHUNK), jnp.int32),
        pltpu.VMEM((NCH32, CHUNK), jnp.float32),
        pltpu.VMEM((NCH32, CHUNK), jnp.float32),
        pltpu.VMEM((CHUNK,), jnp.float32),
        pltpu.VMEM((CHUNK,), jnp.float32),
    ],
)
def _wnorm_kernel(src_hbm, dst_hbm, ew_hbm, dinv_hbm, out_hbm,
                  src_v, dst_v, ew_v, w_v, ds_v, dd_v):
    c = lax.axis_index("c")
    s = lax.axis_index("s")
    wid = c * NS + s

    pltpu.sync_copy(src_hbm.at[wid], src_v)
    pltpu.sync_copy(dst_hbm.at[wid], dst_v)
    pltpu.sync_copy(ew_hbm.at[wid], ew_v)

    def chunk(j, _):
        pltpu.sync_copy(dinv_hbm.at[src_v.at[j]], ds_v)
        pltpu.sync_copy(dinv_hbm.at[dst_v.at[j]], dd_v)
        for q in range(8):
            sl = pl.ds(q * L, L)
            w_v[j, sl] = ds_v[sl] * ew_v[j, sl] * dd_v[sl]
        return 0
    lax.fori_loop(0, NCH32, chunk, 0)
    pltpu.sync_copy(w_v, out_hbm.at[wid])


MCH = 64
NMC = EPT16 // MCH
IB = 8
NB = NMC // IB
RING = 4


@functools.partial(
    pl.kernel,
    out_type=jax.ShapeDtypeStruct((N, NC, CH), jnp.float32),
    mesh=_MESH,
    scratch_types=[
        pltpu.VMEM((2, IB, MCH), jnp.int32),
        pltpu.VMEM((2, IB, MCH), jnp.int32),
        pltpu.VMEM((2, IB, MCH), jnp.float32),
        pltpu.VMEM((RING, MCH, CH), jnp.float32),
        pltpu.VMEM_SHARED((N, CH), jnp.float32),
        pltpu.SemaphoreType.DMA,
        pltpu.SemaphoreType.DMA,
        pltpu.SemaphoreType.DMA,
        pltpu.SemaphoreType.DMA,
    ],
)
def _smv_kernel(xflat_hbm, zeros_hbm, srcl_hbm, srch_hbm, dst_hbm, w_hbm,
                out_hbm, src_v, dst_v, w_v, rows_v, y_s,
                sem_g, sem_s, sem_src, sem_dw):
    c = lax.axis_index("c")
    s = lax.axis_index("s")

    def start_idx(bb, buf):
        @pl.when(c == 0)
        def _():
            pltpu.async_copy(srcl_hbm.at[s, bb], src_v.at[buf], sem_src)

        @pl.when(c == 1)
        def _():
            pltpu.async_copy(srch_hbm.at[s, bb], src_v.at[buf], sem_src)

        pltpu.async_copy(dst_hbm.at[s, bb], dst_v.at[buf], sem_dw)
        pltpu.async_copy(w_hbm.at[s, bb], w_v.at[buf], sem_dw)

    def wait_src():
        pltpu.make_async_copy(
            srcl_hbm.at[s, 0], src_v.at[0], sem_src).wait()

    def wait_dw():
        pltpu.make_async_copy(dst_hbm.at[s, 0], dst_v.at[0], sem_dw).wait()
        pltpu.make_async_copy(w_hbm.at[s, 0], w_v.at[0], sem_dw).wait()

    def start_gather(j):
        buf = lax.rem(j // IB, 2)
        pltpu.async_copy(xflat_hbm.at[src_v.at[buf, lax.rem(j, IB)]],
                         rows_v.at[lax.rem(j, RING)], sem_g)

    def wait_gather():
        pltpu.make_async_copy(
            xflat_hbm.at[src_v.at[0, 0]], rows_v.at[0], sem_g).wait()

    def start_scatter(j):
        buf = lax.rem(j // IB, 2)
        pltpu.async_copy(rows_v.at[lax.rem(j, RING)],
                         y_s.at[dst_v.at[buf, lax.rem(j, IB)]],
                         sem_s, add=True)

    def wait_scatter():
        pltpu.make_async_copy(
            rows_v.at[0], y_s.at[dst_v.at[0, 0]], sem_s).wait()

    @pl.when(c == 0)
    def _():
        pltpu.sync_copy(srcl_hbm.at[s, 0], src_v.at[0])

    @pl.when(c == 1)
    def _():
        pltpu.sync_copy(srch_hbm.at[s, 0], src_v.at[0])

    pltpu.sync_copy(dst_hbm.at[s, 0], dst_v.at[0])
    pltpu.sync_copy(w_hbm.at[s, 0], w_v.at[0])
    start_gather(0)
    start_gather(1)
    pltpu.sync_copy(zeros_hbm, y_s.at[pl.ds(s * RPT, RPT)])
    plsc.subcore_barrier()

    def chunk(j, _):
        bb = j // IB
        pos = lax.rem(j, IB)
        buf = lax.rem(bb, 2)
        r = lax.rem(j, RING)

        @pl.when(j >= 2)
        def _():
            wait_scatter()

        @pl.when(jnp.logical_and(pos == 0, bb >= 1))
        def _():
            wait_dw()

        @pl.when(jnp.logical_and(pos == 1, bb < NB - 1))
        def _():
            start_idx(bb + 1, 1 - buf)

        @pl.when(jnp.logical_and(pos == IB - 2, bb < NB - 1))
        def _():
            wait_src()

        @pl.when(j < NMC - 2)
        def _():
            start_gather(j + 2)

        wait_gather()

        def group(g, _):
            wv16 = w_v[buf, pos, pl.ds(g * L, L)]
            for e16 in range(L):
                wv = lax.broadcast(wv16[e16], (L,))
                row = g * L + e16
                for u in range(CH // L):
                    rows_v[r, row, pl.ds(u * L, L)] = \
                        rows_v[r, row, pl.ds(u * L, L)] * wv
            return 0
        lax.fori_loop(0, MCH // L, group, 0)

        start_scatter(j)
        return 0
    lax.fori_loop(0, NMC, chunk, 0)

    wait_scatter()
    wait_scatter()
    plsc.subcore_barrier()
    pltpu.sync_copy(y_s.at[pl.ds(s * RPT, RPT)],
                    out_hbm.at[pl.ds(s * RPT, RPT), c])


def _dot(a, w):
    return lax.dot_general(a, w, (((1,), (0,)), ((), ())),
                           preferred_element_type=jnp.float32)


def _acc_pair(ref, w_ref, row0):
    return (_dot(ref[:, 0, :], w_ref[pl.ds(row0, CH), :]) +
            _dot(ref[:, 1, :], w_ref[pl.ds(row0 + CH, CH), :]))


def _stage_a_body(x_ref, sx_ref, ssx_ref, h_ref, sh_ref, ssh_ref,
                  wzr_ref, wp_ref, bzr_ref, bp_ref,
                  z_ref, rh_ref, p_ref):
    X = x_ref[...]
    H = h_ref[...]
    zr = (_dot(X, wzr_ref[pl.ds(0, C), :]) +
          _acc_pair(sx_ref, wzr_ref, 256) +
          _acc_pair(ssx_ref, wzr_ref, 512) +
          _dot(H, wzr_ref[pl.ds(768, C), :]) +
          _acc_pair(sh_ref, wzr_ref, 1024) +
          _acc_pair(ssh_ref, wzr_ref, 1280) +
          bzr_ref[...])
    p = (_dot(X, wp_ref[pl.ds(0, C), :]) +
         _acc_pair(sx_ref, wp_ref, 256) +
         _acc_pair(ssx_ref, wp_ref, 512) +
         bp_ref[...])
    Z = jax.nn.sigmoid(zr[:, :C])
    R = jax.nn.sigmoid(zr[:, C:])
    z_ref[...] = Z
    rh_ref[...] = (R * H).reshape(RB, NC, CH)
    p_ref[...] = p


_stage_a_call = pl.pallas_call(
    _stage_a_body,
    grid=(N // RB,),
    in_specs=[
        pl.BlockSpec((RB, C), lambda i: (i, 0)),
        pl.BlockSpec((RB, NC, CH), lambda i: (i, 0, 0)),
        pl.BlockSpec((RB, NC, CH), lambda i: (i, 0, 0)),
        pl.BlockSpec((RB, C), lambda i: (i, 0)),
        pl.BlockSpec((RB, NC, CH), lambda i: (i, 0, 0)),
        pl.BlockSpec((RB, NC, CH), lambda i: (i, 0, 0)),
        pl.BlockSpec((6 * C, 2 * C), lambda i: (0, 0)),
        pl.BlockSpec((3 * C, C), lambda i: (0, 0)),
        pl.BlockSpec((1, 2 * C), lambda i: (0, 0)),
        pl.BlockSpec((1, C), lambda i: (0, 0)),
    ],
    out_specs=[
        pl.BlockSpec((RB, C), lambda i: (i, 0)),
        pl.BlockSpec((RB, NC, CH), lambda i: (i, 0, 0)),
        pl.BlockSpec((RB, C), lambda i: (i, 0)),
    ],
    out_shape=[
        jax.ShapeDtypeStruct((N, C), jnp.float32),
        jax.ShapeDtypeStruct((N, NC, CH), jnp.float32),
        jax.ShapeDtypeStruct((N, C), jnp.float32),
    ],
)


def _final_body(p_ref, rh_ref, srh_ref, ssrh_ref, wc_ref, bc_ref,
                z_ref, h_ref, out_ref):
    acc = (_acc_pair(rh_ref, wc_ref, 0) +
           _acc_pair(srh_ref, wc_ref, 256) +
           _acc_pair(ssrh_ref, wc_ref, 512) +
           p_ref[...] + bc_ref[...])
    Ht = jnp.tanh(acc)
    Z = z_ref[...]
    out_ref[...] = Z * h_ref[...] + (1.0 - Z) * Ht


_final_call = pl.pallas_call(
    _final_body,
    grid=(N // RB,),
    in_specs=[
        pl.BlockSpec((RB, C), lambda i: (i, 0)),
        pl.BlockSpec((RB, NC, CH), lambda i: (i, 0, 0)),
        pl.BlockSpec((RB, NC, CH), lambda i: (i, 0, 0)),
        pl.BlockSpec((RB, NC, CH), lambda i: (i, 0, 0)),
        pl.BlockSpec((3 * C, C), lambda i: (0, 0)),
        pl.BlockSpec((1, C), lambda i: (0, 0)),
        pl.BlockSpec((RB, C), lambda i: (i, 0)),
        pl.BlockSpec((RB, C), lambda i: (i, 0)),
    ],
    out_specs=pl.BlockSpec((RB, C), lambda i: (i, 0)),
    out_shape=jax.ShapeDtypeStruct((N, C), jnp.float32),
)


def _fold(W):
    return jnp.concatenate([W[0] - W[2], -W[1], 2.0 * W[2]], axis=0)


def kernel(X, edge_index, edge_weight, H,
           W_xz, b_xz, W_hz, b_hz, W_xr, b_xr, W_hr, b_hr,
           W_xh, b_xh, W_hh, b_hh):
    src = edge_index[0]
    dst = edge_index[1]
    pad = EPAD - E
    src_p = jnp.concatenate([src, jnp.zeros((pad,), jnp.int32)])
    dst_p = jnp.concatenate([dst, jnp.zeros((pad,), jnp.int32)])
    ew_p = jnp.concatenate([edge_weight, jnp.zeros((pad,), jnp.float32)])

    src32 = src_p.reshape(NC * NS, NCH32, CHUNK)
    dst32 = dst_p.reshape(NC * NS, NCH32, CHUNK)
    ew32 = ew_p.reshape(NC * NS, NCH32, CHUNK)

    deg2 = _deg_kernel(src32, ew32)
    dinv = _dinv_call(deg2.reshape(NC, NPAD // 128, 128)).reshape(NPAD)
    wn = _wnorm_kernel(src32, dst32, ew32, dinv)

    wn16 = wn.reshape(NS, NB, IB, MCH)
    srcl = (2 * src_p).reshape(NS, NB, IB, MCH)
    srch = (2 * src_p + 1).reshape(NS, NB, IB, MCH)
    dst16 = dst_p.reshape(NS, NB, IB, MCH)
    zeros = jnp.zeros((RPT, CH), jnp.float32)

    def smv(xflat):
        return _smv_kernel(xflat, zeros, srcl, srch, dst16, wn16)

    Xf = X.reshape(2 * N, CH)
    Hf = H.reshape(2 * N, CH)
    SX = smv(Xf)
    SSX = smv(SX.reshape(2 * N, CH))
    SH = smv(Hf)
    SSH = smv(SH.reshape(2 * N, CH))

    Wzr = jnp.concatenate([
        jnp.concatenate([_fold(W_xz), _fold(W_hz)], axis=0),
        jnp.concatenate([_fold(W_xr), _fold(W_hr)], axis=0),
    ], axis=1)
    Wp = _fold(W_xh)
    Wc = _fold(W_hh)
    bzr = jnp.concatenate([b_xz + b_hz, b_xr + b_hr]).reshape(1, 2 * C)
    bp = b_xh.reshape(1, C)
    bc = b_hh.reshape(1, C)

    Z, RH2, P = _stage_a_call(X, SX, SSX, H, SH, SSH, Wzr, Wp, bzr, bp)
    SRH = smv(RH2.reshape(2 * N, CH))
    SSRH = smv(SRH.reshape(2 * N, CH))
    return _final_call(P, RH2, SRH, SSRH, Wc, bc, Z, H)


# speedup vs baseline: 2.9378x; 1.0070x over previous
import functools

import jax
import jax.numpy as jnp
from jax import lax
from jax.experimental import pallas as pl
from jax.experimental.pallas import tpu as pltpu
from jax.experimental.pallas import tpu_sc as plsc

N = 10000
E = 160000
C = 256
CH = 128
NC = 2
NS = 16
L = 16
CHUNK = 128

EPAD = 163840
EPT32 = EPAD // (NC * NS)
EPT16 = EPAD // NS
NCH32 = EPT32 // CHUNK
NCH16 = EPT16 // CHUNK
NPAD = 10240
RPT = N // NS
RB = 400

_MESH = plsc.VectorSubcoreMesh(
    core_axis_name="c", subcore_axis_name="s", num_cores=NC, num_subcores=NS)


def _fill_zero(ref, rows, nv):
    def body(i, _):
        r = i // nv
        q = i % nv
        ref[r, pl.ds(q * L, L)] = jnp.zeros((L,), jnp.float32)
        return 0
    lax.fori_loop(0, rows * nv, body, 0)


def _fill_zero_1d(ref, n):
    def body(i, _):
        ref[pl.ds(i * L, L)] = jnp.zeros((L,), jnp.float32)
        return 0
    lax.fori_loop(0, n // L, body, 0)


@functools.partial(
    pl.kernel,
    out_type=jax.ShapeDtypeStruct((NC, NPAD), jnp.float32),
    mesh=_MESH,
    scratch_types=[
        pltpu.VMEM((NCH32, CHUNK), jnp.int32),
        pltpu.VMEM((NCH32, CHUNK), jnp.float32),
        pltpu.VMEM((NPAD // NS,), jnp.float32),
        pltpu.VMEM_SHARED((NPAD,), jnp.float32),
    ],
)
def _deg_kernel(src_hbm, ew_hbm, out_hbm, idx_v, val_v, zero_v, deg_s):
    c = lax.axis_index("c")
    s = lax.axis_index("s")
    wid = c * NS + s
    rpt = NPAD // NS

    _fill_zero_1d(zero_v, rpt)
    pltpu.sync_copy(zero_v, deg_s.at[pl.ds(s * rpt, rpt)])
    plsc.subcore_barrier()

    pltpu.sync_copy(src_hbm.at[wid], idx_v)
    pltpu.sync_copy(ew_hbm.at[wid], val_v)

    def chunk(j, _):
        pltpu.sync_copy(val_v.at[j], deg_s.at[idx_v.at[j]], add=True)
        return 0
    lax.fori_loop(0, NCH32, chunk, 0)
    plsc.subcore_barrier()
    pltpu.sync_copy(deg_s.at[pl.ds(s * rpt, rpt)],
                    out_hbm.at[c, pl.ds(s * rpt, rpt)])


def _dinv_body(deg_ref, dinv_ref):
    d = deg_ref[0] + deg_ref[1]
    dinv_ref[...] = jnp.where(d > 0, lax.rsqrt(d), 0.0)


_dinv_call = pl.pallas_call(
    _dinv_body,
    out_shape=jax.ShapeDtypeStruct((NPAD // 128, 128), jnp.float32),
)


@functools.partial(
    pl.kernel,
    out_type=jax.ShapeDtypeStruct((NC * NS, NCH32, CHUNK), jnp.float32),
    mesh=_MESH,
    scratch_types=[
        pltpu.VMEM((NCH32, CHUNK), jnp.int32),
        pltpu.VMEM((NCH32, CHUNK), jnp.int32),
        pltpu.VMEM((NCH32, CHUNK), jnp.float32),
        pltpu.VMEM((NCH32, CHUNK), jnp.float32),
        pltpu.VMEM((CHUNK,), jnp.float32),
        pltpu.VMEM((CHUNK,), jnp.float32),
    ],
)
def _wnorm_kernel(src_hbm, dst_hbm, ew_hbm, dinv_hbm, out_hbm,
                  src_v, dst_v, ew_v, w_v, ds_v, dd_v):
    c = lax.axis_index("c")
    s = lax.axis_index("s")
    wid = c * NS + s

    pltpu.sync_copy(src_hbm.at[wid], src_v)
    pltpu.sync_copy(dst_hbm.at[wid], dst_v)
    pltpu.sync_copy(ew_hbm.at[wid], ew_v)

    def chunk(j, _):
        pltpu.sync_copy(dinv_hbm.at[src_v.at[j]], ds_v)
        pltpu.sync_copy(dinv_hbm.at[dst_v.at[j]], dd_v)
        for q in range(8):
            sl = pl.ds(q * L, L)
            w_v[j, sl] = ds_v[sl] * ew_v[j, sl] * dd_v[sl]
        return 0
    lax.fori_loop(0, NCH32, chunk, 0)
    pltpu.sync_copy(w_v, out_hbm.at[wid])


MCH = 64
NMC = EPT16 // MCH
IB = 16
NB = NMC // IB
RING = 4


@functools.partial(
    pl.kernel,
    out_type=jax.ShapeDtypeStruct((N, NC, CH), jnp.float32),
    mesh=_MESH,
    scratch_types=[
        pltpu.VMEM((2, IB, MCH), jnp.int32),
        pltpu.VMEM((2, IB, MCH), jnp.int32),
        pltpu.VMEM((2, IB, MCH), jnp.float32),
        pltpu.VMEM((RING, MCH, CH), jnp.float32),
        pltpu.VMEM_SHARED((N, CH), jnp.float32),
        pltpu.SemaphoreType.DMA,
        pltpu.SemaphoreType.DMA,
        pltpu.SemaphoreType.DMA,
        pltpu.SemaphoreType.DMA,
    ],
)
def _smv_kernel(xflat_hbm, zeros_hbm, srcl_hbm, srch_hbm, dst_hbm, w_hbm,
                out_hbm, src_v, dst_v, w_v, rows_v, y_s,
                sem_g, sem_s, sem_src, sem_dw):
    c = lax.axis_index("c")
    s = lax.axis_index("s")

    def start_idx(bb, buf):
        @pl.when(c == 0)
        def _():
            pltpu.async_copy(srcl_hbm.at[s, bb], src_v.at[buf], sem_src)

        @pl.when(c == 1)
        def _():
            pltpu.async_copy(srch_hbm.at[s, bb], src_v.at[buf], sem_src)

        pltpu.async_copy(dst_hbm.at[s, bb], dst_v.at[buf], sem_dw)
        pltpu.async_copy(w_hbm.at[s, bb], w_v.at[buf], sem_dw)

    def wait_src():
        pltpu.make_async_copy(
            srcl_hbm.at[s, 0], src_v.at[0], sem_src).wait()

    def wait_dw():
        pltpu.make_async_copy(dst_hbm.at[s, 0], dst_v.at[0], sem_dw).wait()
        pltpu.make_async_copy(w_hbm.at[s, 0], w_v.at[0], sem_dw).wait()

    def start_gather(j):
        buf = lax.rem(j // IB, 2)
        pltpu.async_copy(xflat_hbm.at[src_v.at[buf, lax.rem(j, IB)]],
                         rows_v.at[lax.rem(j, RING)], sem_g)

    def wait_gather():
        pltpu.make_async_copy(
            xflat_hbm.at[src_v.at[0, 0]], rows_v.at[0], sem_g).wait()

    def start_scatter(j):
        buf = lax.rem(j // IB, 2)
        pltpu.async_copy(rows_v.at[lax.rem(j, RING)],
                         y_s.at[dst_v.at[buf, lax.rem(j, IB)]],
                         sem_s, add=True)

    def wait_scatter():
        pltpu.make_async_copy(
            rows_v.at[0], y_s.at[dst_v.at[0, 0]], sem_s).wait()

    @pl.when(c == 0)
    def _():
        pltpu.sync_copy(srcl_hbm.at[s, 0], src_v.at[0])

    @pl.when(c == 1)
    def _():
        pltpu.sync_copy(srch_hbm.at[s, 0], src_v.at[0])

    pltpu.sync_copy(dst_hbm.at[s, 0], dst_v.at[0])
    pltpu.sync_copy(w_hbm.at[s, 0], w_v.at[0])
    start_gather(0)
    start_gather(1)
    pltpu.sync_copy(zeros_hbm, y_s.at[pl.ds(s * RPT, RPT)])
    plsc.subcore_barrier()

    def chunk(j, _):
        bb = j // IB
        pos = lax.rem(j, IB)
        buf = lax.rem(bb, 2)
        r = lax.rem(j, RING)

        @pl.when(j >= 2)
        def _():
            wait_scatter()

        @pl.when(jnp.logical_and(pos == 0, bb >= 1))
        def _():
            wait_dw()

        @pl.when(jnp.logical_and(pos == 1, bb < NB - 1))
        def _():
            start_idx(bb + 1, 1 - buf)

        @pl.when(jnp.logical_and(pos == IB - 2, bb < NB - 1))
        def _():
            wait_src()

        @pl.when(j < NMC - 2)
        def _():
            start_gather(j + 2)

        wait_gather()

        def group(g, _):
            wv16 = w_v[buf, pos, pl.ds(g * L, L)]
            for e16 in range(L):
                wv = lax.broadcast(wv16[e16], (L,))
                row = g * L + e16
                for u in range(CH // L):
                    rows_v[r, row, pl.ds(u * L, L)] = \
                        rows_v[r, row, pl.ds(u * L, L)] * wv
            return 0
        lax.fori_loop(0, MCH // L, group, 0)

        start_scatter(j)
        return 0
    lax.fori_loop(0, NMC, chunk, 0)

    wait_scatter()
    wait_scatter()
    plsc.subcore_barrier()
    pltpu.sync_copy(y_s.at[pl.ds(s * RPT, RPT)],
                    out_hbm.at[pl.ds(s * RPT, RPT), c])


def _dot(a, w):
    return lax.dot_general(a, w, (((1,), (0,)), ((), ())),
                           preferred_element_type=jnp.float32)


def _acc_pair(ref, w_ref, row0):
    return (_dot(ref[:, 0, :], w_ref[pl.ds(row0, CH), :]) +
            _dot(ref[:, 1, :], w_ref[pl.ds(row0 + CH, CH), :]))


def _stage_a_body(x_ref, sx_ref, ssx_ref, h_ref, sh_ref, ssh_ref,
                  wzr_ref, wp_ref, bzr_ref, bp_ref,
                  z_ref, rh_ref, p_ref):
    X = x_ref[...]
    H = h_ref[...]
    zr = (_dot(X, wzr_ref[pl.ds(0, C), :]) +
          _acc_pair(sx_ref, wzr_ref, 256) +
          _acc_pair(ssx_ref, wzr_ref, 512) +
          _dot(H, wzr_ref[pl.ds(768, C), :]) +
          _acc_pair(sh_ref, wzr_ref, 1024) +
          _acc_pair(ssh_ref, wzr_ref, 1280) +
          bzr_ref[...])
    p = (_dot(X, wp_ref[pl.ds(0, C), :]) +
         _acc_pair(sx_ref, wp_ref, 256) +
         _acc_pair(ssx_ref, wp_ref, 512) +
         bp_ref[...])
    Z = jax.nn.sigmoid(zr[:, :C])
    R = jax.nn.sigmoid(zr[:, C:])
    z_ref[...] = Z
    rh_ref[...] = (R * H).reshape(RB, NC, CH)
    p_ref[...] = p


_stage_a_call = pl.pallas_call(
    _stage_a_body,
    grid=(N // RB,),
    in_specs=[
        pl.BlockSpec((RB, C), lambda i: (i, 0)),
        pl.BlockSpec((RB, NC, CH), lambda i: (i, 0, 0)),
        pl.BlockSpec((RB, NC, CH), lambda i: (i, 0, 0)),
        pl.BlockSpec((RB, C), lambda i: (i, 0)),
        pl.BlockSpec((RB, NC, CH), lambda i: (i, 0, 0)),
        pl.BlockSpec((RB, NC, CH), lambda i: (i, 0, 0)),
        pl.BlockSpec((6 * C, 2 * C), lambda i: (0, 0)),
        pl.BlockSpec((3 * C, C), lambda i: (0, 0)),
        pl.BlockSpec((1, 2 * C), lambda i: (0, 0)),
        pl.BlockSpec((1, C), lambda i: (0, 0)),
    ],
    out_specs=[
        pl.BlockSpec((RB, C), lambda i: (i, 0)),
        pl.BlockSpec((RB, NC, CH), lambda i: (i, 0, 0)),
        pl.BlockSpec((RB, C), lambda i: (i, 0)),
    ],
    out_shape=[
        jax.ShapeDtypeStruct((N, C), jnp.float32),
        jax.ShapeDtypeStruct((N, NC, CH), jnp.float32),
        jax.ShapeDtypeStruct((N, C), jnp.float32),
    ],
)


def _final_body(p_ref, rh_ref, srh_ref, ssrh_ref, wc_ref, bc_ref,
                z_ref, h_ref, out_ref):
    acc = (_acc_pair(rh_ref, wc_ref, 0) +
           _acc_pair(srh_ref, wc_ref, 256) +
           _acc_pair(ssrh_ref, wc_ref, 512) +
           p_ref[...] + bc_ref[...])
    Ht = jnp.tanh(acc)
    Z = z_ref[...]
    out_ref[...] = Z * h_ref[...] + (1.0 - Z) * Ht


_final_call = pl.pallas_call(
    _final_body,
    grid=(N // RB,),
    in_specs=[
        pl.BlockSpec((RB, C), lambda i: (i, 0)),
        pl.BlockSpec((RB, NC, CH), lambda i: (i, 0, 0)),
        pl.BlockSpec((RB, NC, CH), lambda i: (i, 0, 0)),
        pl.BlockSpec((RB, NC, CH), lambda i: (i, 0, 0)),
        pl.BlockSpec((3 * C, C), lambda i: (0, 0)),
        pl.BlockSpec((1, C), lambda i: (0, 0)),
        pl.BlockSpec((RB, C), lambda i: (i, 0)),
        pl.BlockSpec((RB, C), lambda i: (i, 0)),
    ],
    out_specs=pl.BlockSpec((RB, C), lambda i: (i, 0)),
    out_shape=jax.ShapeDtypeStruct((N, C), jnp.float32),
)


def _fold(W):
    return jnp.concatenate([W[0] - W[2], -W[1], 2.0 * W[2]], axis=0)


def kernel(X, edge_index, edge_weight, H,
           W_xz, b_xz, W_hz, b_hz, W_xr, b_xr, W_hr, b_hr,
           W_xh, b_xh, W_hh, b_hh):
    src = edge_index[0]
    dst = edge_index[1]
    pad = EPAD - E
    src_p = jnp.concatenate([src, jnp.zeros((pad,), jnp.int32)])
    dst_p = jnp.concatenate([dst, jnp.zeros((pad,), jnp.int32)])
    ew_p = jnp.concatenate([edge_weight, jnp.zeros((pad,), jnp.float32)])

    src32 = src_p.reshape(NC * NS, NCH32, CHUNK)
    dst32 = dst_p.reshape(NC * NS, NCH32, CHUNK)
    ew32 = ew_p.reshape(NC * NS, NCH32, CHUNK)

    deg2 = _deg_kernel(src32, ew32)
    dinv = _dinv_call(deg2.reshape(NC, NPAD // 128, 128)).reshape(NPAD)
    wn = _wnorm_kernel(src32, dst32, ew32, dinv)

    wn16 = wn.reshape(NS, NB, IB, MCH)
    srcl = (2 * src_p).reshape(NS, NB, IB, MCH)
    srch = (2 * src_p + 1).reshape(NS, NB, IB, MCH)
    dst16 = dst_p.reshape(NS, NB, IB, MCH)
    zeros = jnp.zeros((RPT, CH), jnp.float32)

    def smv(xflat):
        return _smv_kernel(xflat, zeros, srcl, srch, dst16, wn16)

    Xf = X.reshape(2 * N, CH)
    Hf = H.reshape(2 * N, CH)
    SX = smv(Xf)
    SSX = smv(SX.reshape(2 * N, CH))
    SH = smv(Hf)
    SSH = smv(SH.reshape(2 * N, CH))

    Wzr = jnp.concatenate([
        jnp.concatenate([_fold(W_xz), _fold(W_hz)], axis=0),
        jnp.concatenate([_fold(W_xr), _fold(W_hr)], axis=0),
    ], axis=1)
    Wp = _fold(W_xh)
    Wc = _fold(W_hh)
    bzr = jnp.concatenate([b_xz + b_hz, b_xr + b_hr]).reshape(1, 2 * C)
    bp = b_xh.reshape(1, C)
    bc = b_hh.reshape(1, C)

    Z, RH2, P = _stage_a_call(X, SX, SSX, H, SH, SSH, Wzr, Wp, bzr, bp)
    SRH = smv(RH2.reshape(2 * N, CH))
    SSRH = smv(SRH.reshape(2 * N, CH))
    return _final_call(P, RH2, SRH, SSRH, Wc, bc, Z, H)
